# baseline (speedup 1.0000x reference)
import functools

import jax
import jax.numpy as jnp
from jax import lax
from jax.experimental import pallas as pl
from jax.experimental.pallas import tpu as pltpu
from jax.experimental.pallas import tpu_sc as plsc

_D = 64
_W = 128
_CHUNK = 128
_LANES = 16
_NBUF = 4
_ABLK = 16384
_ABLK_LOG2 = 14
_SCALE = 8.0


def _transpose_block(in_ref, out_ref):
    x = in_ref[...] * _SCALE
    out_ref[:, 0:_D] = x[:, 0 : _ABLK // 2].T
    out_ref[:, _D:_W] = x[:, _ABLK // 2 : _ABLK].T


@functools.lru_cache(maxsize=None)
def _make_table_transpose(vocab):
    grid = (vocab + _ABLK - 1) // _ABLK
    rows = grid * (_ABLK // 2)
    return pl.pallas_call(
        _transpose_block,
        grid=(grid,),
        in_specs=[pl.BlockSpec((_D, _ABLK), lambda g: (0, g))],
        out_specs=pl.BlockSpec((_ABLK // 2, _W), lambda g: (g, 0)),
        out_shape=jax.ShapeDtypeStruct((rows, _W), jnp.float32),
    )


@functools.lru_cache(maxsize=None)
def _make_sc_gather(seq_len, batch, num_workers):
    mesh = plsc.VectorSubcoreMesh(core_axis_name="c", subcore_axis_name="s")
    num_cores = 2
    outer_n = seq_len // _NBUF

    @functools.partial(
        pl.kernel,
        out_type=jax.ShapeDtypeStruct((seq_len * _D, batch), jnp.float32),
        mesh=mesh,
        scratch_types=[
            pltpu.VMEM((seq_len, _CHUNK), jnp.int32),
            pltpu.VMEM((_NBUF, _CHUNK), jnp.int32),
            *[pltpu.VMEM((_CHUNK, _W), jnp.float32) for _ in range(_NBUF)],
            *[pltpu.VMEM((_D, _CHUNK), jnp.float32) for _ in range(_NBUF)],
            pltpu.SemaphoreType.DMA,
            *[pltpu.SemaphoreType.DMA for _ in range(2 * _NBUF)],
        ],
        compiler_params=pltpu.CompilerParams(needs_layout_passes=False),
    )
    def sc_gather(ids_hbm, table_hbm, out_hbm, idx_v, pidx, *scr):
        gbufs = scr[:_NBUF]
        obufs = scr[_NBUF : 2 * _NBUF]
        sem0 = scr[2 * _NBUF]
        gsems = scr[2 * _NBUF + 1 : 3 * _NBUF + 1]
        osems = scr[3 * _NBUF + 1 : 4 * _NBUF + 1]

        wid = lax.axis_index("s") * num_cores + lax.axis_index("c")
        bcol = wid * _CHUNK
        pltpu.async_copy(
            ids_hbm.at[:, pl.ds(bcol, _CHUNK)], idx_v, sem0
        ).wait()

        def fill_pidx(b, j):
            for c in range(_CHUNK // _LANES):
                sl = pl.ds(c * _LANES, _LANES)
                tid = idx_v[j, sl]
                pidx[b, sl] = (
                    lax.shift_left(
                        lax.shift_right_logical(tid, _ABLK_LOG2), _ABLK_LOG2 - 1
                    )
                    + (tid & (_ABLK // 2 - 1))
                )

        for b in range(_NBUF):
            fill_pidx(b, b)
            pltpu.async_copy(table_hbm.at[pidx.at[b]], gbufs[b], gsems[b])

        lane = lax.broadcasted_iota(jnp.int32, (_LANES,), 0)

        def outer(g, carry):
            for b in range(_NBUF):
                j = g * _NBUF + b
                pltpu.make_async_copy(
                    table_hbm.at[pidx.at[b]], gbufs[b], gsems[b]
                ).wait()
                @pl.when(g > 0)
                def _():
                    pltpu.make_async_copy(
                        obufs[b],
                        out_hbm.at[pl.ds(0, _D), pl.ds(bcol, _CHUNK)],
                        osems[b],
                    ).wait()

                gb = gbufs[b]
                ob = obufs[b]

                def select_rows(t, c):
                    tid = idx_v[j, pl.ds(t * _LANES, _LANES)]
                    halfsel = jnp.where((tid & (_ABLK - 1)) >= _ABLK // 2, _D, 0)
                    rowv = t * _LANES + lane

                    @plsc.parallel_loop(0, _D, 1, unroll=8)
                    def per_dim(d0):
                        dvec = (d0 + lane) & (_D - 1)
                        v = plsc.load_gather(gb, [rowv, halfsel + dvec])
                        plsc.store_scatter(ob, [dvec, rowv], v)

                    return c

                lax.fori_loop(0, _CHUNK // _LANES, select_rows, 0)

                pltpu.async_copy(
                    ob,
                    out_hbm.at[pl.ds(j * _D, _D), pl.ds(bcol, _CHUNK)],
                    osems[b],
                )

                @pl.when(g < outer_n - 1)
                def _():
                    fill_pidx(b, j + _NBUF)
                    pltpu.async_copy(
                        table_hbm.at[pidx.at[b]], gbufs[b], gsems[b]
                    )

            return carry

        lax.fori_loop(0, outer_n, outer, 0)

        for b in range(_NBUF):
            pltpu.make_async_copy(
                obufs[b],
                out_hbm.at[pl.ds(0, _D), pl.ds(bcol, _CHUNK)],
                osems[b],
            ).wait()

    return sc_gather


def kernel(input_ids, token_table):
    batch, seq_len = input_ids.shape
    vocab, d_model = token_table.shape
    assert d_model == _D

    ids_t = input_ids.astype(jnp.int32).T
    table_t = token_table.T

    table2 = _make_table_transpose(vocab)(table_t)
    out2 = _make_sc_gather(seq_len, batch, 32)(ids_t, table2)
    return out2.reshape(seq_len, _D, batch).transpose(2, 0, 1)


# speedup vs baseline: 1.0288x; 1.0288x over previous
import functools

import jax
import jax.numpy as jnp
from jax import lax
from jax.experimental import pallas as pl
from jax.experimental.pallas import tpu as pltpu
from jax.experimental.pallas import tpu_sc as plsc

_D = 64
_W = 128
_CHUNK = 128
_LANES = 16
_NBUF = 4
_ABLK = 32768
_ABLK_LOG2 = 15
_SCALE = 8.0


def _transpose_block(in_ref, out_ref):
    x = in_ref[...] * _SCALE
    out_ref[:, 0:_D] = x[:, 0 : _ABLK // 2].T
    out_ref[:, _D:_W] = x[:, _ABLK // 2 : _ABLK].T


@functools.lru_cache(maxsize=None)
def _make_table_transpose(vocab):
    grid = (vocab + _ABLK - 1) // _ABLK
    rows = grid * (_ABLK // 2)
    return pl.pallas_call(
        _transpose_block,
        grid=(grid,),
        in_specs=[pl.BlockSpec((_D, _ABLK), lambda g: (0, g))],
        out_specs=pl.BlockSpec((_ABLK // 2, _W), lambda g: (g, 0)),
        out_shape=jax.ShapeDtypeStruct((rows, _W), jnp.float32),
    )


@functools.lru_cache(maxsize=None)
def _make_sc_gather(seq_len, batch, num_workers):
    mesh = plsc.VectorSubcoreMesh(core_axis_name="c", subcore_axis_name="s")
    num_cores = 2
    outer_n = seq_len // _NBUF

    @functools.partial(
        pl.kernel,
        out_type=jax.ShapeDtypeStruct((seq_len * _D, batch), jnp.float32),
        mesh=mesh,
        scratch_types=[
            pltpu.VMEM((seq_len, _CHUNK), jnp.int32),
            pltpu.VMEM((_NBUF, _CHUNK), jnp.int32),
            *[pltpu.VMEM((_CHUNK, _W), jnp.float32) for _ in range(_NBUF)],
            *[pltpu.VMEM((_D, _CHUNK), jnp.float32) for _ in range(_NBUF)],
            pltpu.SemaphoreType.DMA,
            *[pltpu.SemaphoreType.DMA for _ in range(2 * _NBUF)],
        ],
        compiler_params=pltpu.CompilerParams(needs_layout_passes=False),
    )
    def sc_gather(ids_hbm, table_hbm, out_hbm, idx_v, pidx, *scr):
        gbufs = scr[:_NBUF]
        obufs = scr[_NBUF : 2 * _NBUF]
        sem0 = scr[2 * _NBUF]
        gsems = scr[2 * _NBUF + 1 : 3 * _NBUF + 1]
        osems = scr[3 * _NBUF + 1 : 4 * _NBUF + 1]

        wid = lax.axis_index("s") * num_cores + lax.axis_index("c")
        bcol = wid * _CHUNK
        pltpu.async_copy(
            ids_hbm.at[:, pl.ds(bcol, _CHUNK)], idx_v, sem0
        ).wait()

        def fill_pidx(b, j):
            for c in range(_CHUNK // _LANES):
                sl = pl.ds(c * _LANES, _LANES)
                tid = idx_v[j, sl]
                pidx[b, sl] = (
                    lax.shift_left(
                        lax.shift_right_logical(tid, _ABLK_LOG2), _ABLK_LOG2 - 1
                    )
                    + (tid & (_ABLK // 2 - 1))
                )

        for b in range(_NBUF):
            fill_pidx(b, b)
            pltpu.async_copy(table_hbm.at[pidx.at[b]], gbufs[b], gsems[b])

        lane = lax.broadcasted_iota(jnp.int32, (_LANES,), 0)

        def outer(g, carry):
            for b in range(_NBUF):
                j = g * _NBUF + b
                pltpu.make_async_copy(
                    table_hbm.at[pidx.at[b]], gbufs[b], gsems[b]
                ).wait()
                @pl.when(g > 0)
                def _():
                    pltpu.make_async_copy(
                        obufs[b],
                        out_hbm.at[pl.ds(0, _D), pl.ds(bcol, _CHUNK)],
                        osems[b],
                    ).wait()

                gb = gbufs[b]
                ob = obufs[b]

                def select_rows(t, c):
                    tid = idx_v[j, pl.ds(t * _LANES, _LANES)]
                    halfsel = jnp.where((tid & (_ABLK - 1)) >= _ABLK // 2, _D, 0)
                    rowv = t * _LANES + lane

                    @plsc.parallel_loop(0, _D, 1, unroll=8)
                    def per_dim(d0):
                        dvec = (d0 + lane) & (_D - 1)
                        v = plsc.load_gather(gb, [rowv, halfsel + dvec])
                        plsc.store_scatter(ob, [dvec, rowv], v)

                    return c

                lax.fori_loop(0, _CHUNK // _LANES, select_rows, 0)

                pltpu.async_copy(
                    ob,
                    out_hbm.at[pl.ds(j * _D, _D), pl.ds(bcol, _CHUNK)],
                    osems[b],
                )

                @pl.when(g < outer_n - 1)
                def _():
                    fill_pidx(b, j + _NBUF)
                    pltpu.async_copy(
                        table_hbm.at[pidx.at[b]], gbufs[b], gsems[b]
                    )

            return carry

        lax.fori_loop(0, outer_n, outer, 0)

        for b in range(_NBUF):
            pltpu.make_async_copy(
                obufs[b],
                out_hbm.at[pl.ds(0, _D), pl.ds(bcol, _CHUNK)],
                osems[b],
            ).wait()

    return sc_gather


def kernel(input_ids, token_table):
    batch, seq_len = input_ids.shape
    vocab, d_model = token_table.shape
    assert d_model == _D

    ids_t = input_ids.astype(jnp.int32).T
    table_t = token_table.T

    table2 = _make_table_transpose(vocab)(table_t)
    out2 = _make_sc_gather(seq_len, batch, 32)(ids_t, table2)
    return out2.reshape(seq_len, _D, batch).transpose(2, 0, 1)
